# baseline (speedup 1.0000x reference)
import jax
import jax.numpy as jnp
from jax.experimental import pallas as pl

N = 50000
SLOPE = 0.2


def _elu_body(h_ref, o_ref):
    h = h_ref[...]
    o_ref[...] = jnp.where(h > 0, h, jnp.exp(h) - 1.0)


def kernel(m_sim, d_sim, node_type, edge_index, W_m, W_d):
    z_m = m_sim @ W_m
    z_d = d_sim @ W_d
    z = jnp.where((node_type == 0)[:, None], z_m, z_d)
    src = edge_index[0]
    dst = edge_index[1]
    e = jnp.sum(z[src] * z[dst], axis=1)
    e = jnp.where(e >= 0, e, SLOPE * e)
    emax = jax.ops.segment_max(e, dst, num_segments=N)
    emax = jnp.where(jnp.isfinite(emax), emax, 0.0)
    ex = jnp.exp(e - emax[dst])
    denom = jax.ops.segment_sum(ex, dst, num_segments=N)
    denom_safe = jnp.where(denom > 0, denom, 1.0)
    alpha = ex / denom_safe[dst]
    h = jax.ops.segment_sum(alpha[:, None] * z[src], dst, num_segments=N)
    grid = (N // 400,)
    h = pl.pallas_call(
        _elu_body,
        grid=grid,
        in_specs=[pl.BlockSpec((400, 64), lambda i: (i, 0))],
        out_specs=pl.BlockSpec((400, 64), lambda i: (i, 0)),
        out_shape=jax.ShapeDtypeStruct((N, 64), jnp.float32),
    )(h)
    return h


# speedup vs baseline: 2.9911x; 2.9911x over previous
import functools

import jax
import jax.numpy as jnp
from jax import lax
from jax.experimental import pallas as pl
from jax.experimental.pallas import tpu as pltpu
from jax.experimental.pallas import tpu_sc as plsc

N = 50000
NPAD = 50176
SLICE = NPAD // 16
F = 64
HF = 32
E = 800000
EPAD = 819200
EW = EPAD // 32
EW3 = EPAD // 16
C = 128
SLOPE = 0.2
BLK_A = 512
BLK_B = 400

_mesh = plsc.VectorSubcoreMesh(core_axis_name="c", subcore_axis_name="s")


def _iota16():
    return lax.iota(jnp.int32, 16)


def _zbody(m_ref, d_ref, nt_ref, wm_ref, wd_ref, z_ref):
    zm = jnp.dot(m_ref[...], wm_ref[...], preferred_element_type=jnp.float32)
    zd = jnp.dot(d_ref[...], wd_ref[...], preferred_element_type=jnp.float32)
    z_ref[...] = jnp.where(nt_ref[...] == 0, zm, zd)


def _compute_z(m_p, d_p, nt_p, W_m, W_d):
    md = m_p.shape[1]
    dd = d_p.shape[1]
    return pl.pallas_call(
        _zbody,
        grid=(NPAD // BLK_A,),
        in_specs=[
            pl.BlockSpec((BLK_A, md), lambda i: (i, 0)),
            pl.BlockSpec((BLK_A, dd), lambda i: (i, 0)),
            pl.BlockSpec((BLK_A, 1), lambda i: (i, 0)),
            pl.BlockSpec((md, F), lambda i: (0, 0)),
            pl.BlockSpec((dd, F), lambda i: (0, 0)),
        ],
        out_specs=pl.BlockSpec((BLK_A, F), lambda i: (i, 0)),
        out_shape=jax.ShapeDtypeStruct((NPAD, F), jnp.float32),
    )(m_p, d_p, nt_p, W_m, W_d)


@functools.partial(
    pl.kernel,
    out_type=[
        jax.ShapeDtypeStruct((EPAD,), jnp.float32),
        jax.ShapeDtypeStruct((2, NPAD), jnp.float32),
    ],
    mesh=_mesh,
    compiler_params=pltpu.CompilerParams(use_tc_tiling_on_sc=False, needs_layout_passes=False),
    scratch_types=[
        pltpu.VMEM((C,), jnp.int32),
        pltpu.VMEM((C,), jnp.int32),
        pltpu.VMEM((C, F), jnp.float32),
        pltpu.VMEM((C, F), jnp.float32),
        pltpu.VMEM((C,), jnp.float32),
        pltpu.VMEM((NPAD,), jnp.float32),
        pltpu.VMEM((SLICE,), jnp.float32),
        pltpu.VMEM((SLICE,), jnp.float32),
        pltpu.VMEM_SHARED((16, NPAD), jnp.float32),
        pltpu.SemaphoreType.DMA,
        pltpu.SemaphoreType.DMA,
    ],
)
def _sc_pass1(z_hbm, src_hbm, dst_hbm, e_hbm, emax_hbm,
              src_v, dst_v, zs_v, zd_v, e_v, emax_p, acc_v, tmp_v, sp, sem1, sem2):
    cid = lax.axis_index("c")
    sid = lax.axis_index("s")
    base = (cid * 16 + sid) * EW
    iota = _iota16()

    def init_body(i, _):
        emax_p[pl.ds(i * 16, 16)] = jnp.full((16,), -jnp.inf, jnp.float32)
        return 0
    lax.fori_loop(0, NPAD // 16, init_body, 0)

    def chunk_body(ci, _):
        off = base + ci * C
        pltpu.sync_copy(src_hbm.at[pl.ds(off, C)], src_v)
        pltpu.sync_copy(dst_hbm.at[pl.ds(off, C)], dst_v)
        cp1 = pltpu.async_copy(z_hbm.at[src_v], zs_v, sem1)
        cp2 = pltpu.async_copy(z_hbm.at[dst_v], zd_v, sem2)
        cp1.wait()
        cp2.wait()
        for g in range(C // 16):
            rows = iota + (g * 16)

            def dot_body(f, acc):
                fs = jnp.full((16,), f, jnp.int32)
                sv = plsc.load_gather(zs_v, [rows, fs])
                dv = plsc.load_gather(zd_v, [rows, fs])
                return acc + sv * dv
            acc = lax.fori_loop(0, F, dot_body, jnp.zeros((16,), jnp.float32))
            e16 = jnp.where(acc >= 0.0, acc, SLOPE * acc)
            e_v[pl.ds(g * 16, 16)] = e16
            dv16 = dst_v[pl.ds(g * 16, 16)]

            def wbody(go):
                cur = plsc.load_gather(emax_p, [dv16])
                need = e16 > cur
                plsc.store_scatter(emax_p, [dv16], jnp.maximum(cur, e16), mask=need)
                return jnp.any(need)
            lax.while_loop(lambda go: go, wbody, jnp.bool_(True))
        pltpu.sync_copy(e_v, e_hbm.at[pl.ds(off, C)])
        return 0
    lax.fori_loop(0, EW // C, chunk_body, 0)

    pltpu.sync_copy(emax_p, sp.at[sid])
    plsc.subcore_barrier()
    roff = sid * SLICE
    pltpu.sync_copy(sp.at[0, pl.ds(roff, SLICE)], acc_v)

    def comb_body(t, _):
        pltpu.sync_copy(sp.at[t, pl.ds(roff, SLICE)], tmp_v)

        def vb(i, _):
            s = pl.ds(i * 16, 16)
            acc_v[s] = jnp.maximum(acc_v[s], tmp_v[s])
            return 0
        lax.fori_loop(0, SLICE // 16, vb, 0)
        return 0
    lax.fori_loop(1, 16, comb_body, 0)
    pltpu.sync_copy(acc_v, emax_hbm.at[cid, pl.ds(roff, SLICE)])


@functools.partial(
    pl.kernel,
    out_type=jax.ShapeDtypeStruct((EPAD,), jnp.float32),
    mesh=_mesh,
    compiler_params=pltpu.CompilerParams(use_tc_tiling_on_sc=False, needs_layout_passes=False),
    scratch_types=[
        pltpu.VMEM((C,), jnp.float32),
        pltpu.VMEM((C,), jnp.int32),
        pltpu.VMEM((C,), jnp.float32),
        pltpu.VMEM((NPAD,), jnp.float32),
        pltpu.VMEM((NPAD,), jnp.float32),
        pltpu.VMEM((SLICE,), jnp.float32),
        pltpu.VMEM((SLICE,), jnp.float32),
        pltpu.VMEM_SHARED((NPAD,), jnp.float32),
    ],
)
def _sc_pass2(e_hbm, dst_hbm, emax2_hbm, al_hbm,
              e_v, dst_v, w_v, emax_p, den_p, tmp_v, zero_v, den_sp):
    cid = lax.axis_index("c")
    sid = lax.axis_index("s")
    roff = sid * SLICE

    pltpu.sync_copy(emax2_hbm.at[0], emax_p)

    def mslice(j, _):
        pltpu.sync_copy(emax2_hbm.at[1, pl.ds(j * SLICE, SLICE)], tmp_v)

        def vb(i, _):
            s = pl.ds(i * 16, 16)
            k = pl.ds(j * SLICE + i * 16, 16)
            emax_p[k] = jnp.maximum(emax_p[k], tmp_v[s])
            return 0
        lax.fori_loop(0, SLICE // 16, vb, 0)
        return 0
    lax.fori_loop(0, 16, mslice, 0)

    def zb(i, _):
        zero_v[pl.ds(i * 16, 16)] = jnp.zeros((16,), jnp.float32)
        return 0
    lax.fori_loop(0, SLICE // 16, zb, 0)
    pltpu.sync_copy(zero_v, den_sp.at[pl.ds(roff, SLICE)])
    plsc.subcore_barrier()

    base1 = sid * EW3

    def den_body(ci, _):
        off = base1 + ci * C
        pltpu.sync_copy(e_hbm.at[pl.ds(off, C)], e_v)
        pltpu.sync_copy(dst_hbm.at[pl.ds(off, C)], dst_v)
        for g in range(C // 16):
            s = pl.ds(g * 16, 16)
            m16 = plsc.load_gather(emax_p, [dst_v[s]])
            w_v[s] = jnp.exp(e_v[s] - m16)
        pltpu.sync_copy(w_v, den_sp.at[dst_v], add=True)
        return 0
    lax.fori_loop(0, EW3 // C, den_body, 0)
    plsc.subcore_barrier()

    pltpu.sync_copy(den_sp, den_p)

    def sb(i, _):
        s = pl.ds(i * 16, 16)
        d = den_p[s]
        den_p[s] = jnp.where(d > 0.0, d, 1.0)
        return 0
    lax.fori_loop(0, NPAD // 16, sb, 0)

    base2 = (cid * 16 + sid) * EW

    def al_body(ci, _):
        off = base2 + ci * C
        pltpu.sync_copy(e_hbm.at[pl.ds(off, C)], e_v)
        pltpu.sync_copy(dst_hbm.at[pl.ds(off, C)], dst_v)
        for g in range(C // 16):
            s = pl.ds(g * 16, 16)
            m16 = plsc.load_gather(emax_p, [dst_v[s]])
            d16 = plsc.load_gather(den_p, [dst_v[s]])
            w_v[s] = jnp.exp(e_v[s] - m16) / d16
        pltpu.sync_copy(w_v, al_hbm.at[pl.ds(off, C)])
        return 0
    lax.fori_loop(0, EW // C, al_body, 0)


@functools.partial(
    pl.kernel,
    out_type=jax.ShapeDtypeStruct((2, NPAD, HF), jnp.float32),
    mesh=_mesh,
    compiler_params=pltpu.CompilerParams(use_tc_tiling_on_sc=False, needs_layout_passes=False),
    scratch_types=[
        pltpu.VMEM((C,), jnp.int32),
        pltpu.VMEM((C,), jnp.int32),
        pltpu.VMEM((C,), jnp.int32),
        pltpu.VMEM((C,), jnp.float32),
        pltpu.VMEM((C, HF), jnp.float32),
        pltpu.VMEM((448, HF), jnp.float32),
        pltpu.VMEM_SHARED((NPAD, HF), jnp.float32),
        pltpu.SemaphoreType.DMA,
    ],
)
def _sc_pass3(z2_hbm, src_hbm, dst_hbm, al_hbm, h2_hbm,
              src_v, dst_v, idx2_v, al_v, zr_v, rb_v, hsp, sem):
    cid = lax.axis_index("c")
    sid = lax.axis_index("s")
    base = sid * EW3
    iota = _iota16()
    roff = sid * SLICE

    def zrow2(i, _):
        def zcol(k, _):
            rb_v[i, pl.ds(k * 16, 16)] = jnp.zeros((16,), jnp.float32)
            return 0
        lax.fori_loop(0, HF // 16, zcol, 0)
        return 0
    lax.fori_loop(0, 448, zrow2, 0)

    def zcp(i, _):
        pltpu.sync_copy(rb_v, hsp.at[pl.ds(roff + i * 448, 448)])
        return 0
    lax.fori_loop(0, SLICE // 448, zcp, 0)
    plsc.subcore_barrier()

    def chunk_body(ci, _):
        off = base + ci * C
        pltpu.sync_copy(src_hbm.at[pl.ds(off, C)], src_v)
        pltpu.sync_copy(dst_hbm.at[pl.ds(off, C)], dst_v)
        pltpu.sync_copy(al_hbm.at[pl.ds(off, C)], al_v)
        for g in range(C // 16):
            s = pl.ds(g * 16, 16)
            idx2_v[s] = src_v[s] * 2 + cid
        pltpu.async_copy(z2_hbm.at[idx2_v], zr_v, sem).wait()
        for g in range(C // 16):
            s = pl.ds(g * 16, 16)
            rows = iota + (g * 16)
            al16 = al_v[s]

            def scale_body(f, _):
                fs = jnp.full((16,), f, jnp.int32)
                col = plsc.load_gather(zr_v, [rows, fs])
                plsc.store_scatter(zr_v, [rows, fs], col * al16)
                return 0
            lax.fori_loop(0, HF, scale_body, 0)
        pltpu.sync_copy(zr_v, hsp.at[dst_v], add=True)
        return 0
    lax.fori_loop(0, EW3 // C, chunk_body, 0)

    plsc.subcore_barrier()

    def outcp(i, _):
        r0 = roff + i * 448
        pltpu.sync_copy(hsp.at[pl.ds(r0, 448)], rb_v)
        pltpu.sync_copy(rb_v, h2_hbm.at[cid, pl.ds(r0, 448)])
        return 0
    lax.fori_loop(0, SLICE // 448, outcp, 0)


def _elu_body(a_ref, b_ref, o_ref):
    h = jnp.concatenate([a_ref[0], b_ref[0]], axis=1)
    o_ref[...] = jnp.where(h > 0, h, jnp.exp(h) - 1.0)


def _elu_merge(h2):
    return pl.pallas_call(
        _elu_body,
        grid=(N // BLK_B,),
        in_specs=[
            pl.BlockSpec((1, BLK_B, HF), lambda i: (0, i, 0)),
            pl.BlockSpec((1, BLK_B, HF), lambda i: (1, i, 0)),
        ],
        out_specs=pl.BlockSpec((BLK_B, F), lambda i: (i, 0)),
        out_shape=jax.ShapeDtypeStruct((N, F), jnp.float32),
    )(h2, h2)


def kernel(m_sim, d_sim, node_type, edge_index, W_m, W_d):
    m_p = jnp.pad(m_sim, ((0, NPAD - N), (0, 0)))
    d_p = jnp.pad(d_sim, ((0, NPAD - N), (0, 0)))
    nt_p = jnp.pad(node_type.astype(jnp.int32), (0, NPAD - N)).reshape(NPAD, 1)

    src = edge_index[0].astype(jnp.int32)
    dst = edge_index[1].astype(jnp.int32)
    pad_ids = (jnp.arange(EPAD - E, dtype=jnp.int32) % (NPAD - N)) + N
    srcp = jnp.concatenate([src, pad_ids])
    dstp = jnp.concatenate([dst, pad_ids])

    z = _compute_z(m_p, d_p, nt_p, W_m, W_d)
    z2 = z.reshape(2 * NPAD, HF)

    e, emax2 = _sc_pass1(z, srcp, dstp)
    alpha = _sc_pass2(e, dstp, emax2)
    h2 = _sc_pass3(z2, srcp, dstp, alpha)
    return _elu_merge(h2)


# speedup vs baseline: 3.4336x; 1.1480x over previous
import functools

import jax
import jax.numpy as jnp
from jax import lax
from jax.experimental import pallas as pl
from jax.experimental.pallas import tpu as pltpu
from jax.experimental.pallas import tpu_sc as plsc

N = 50000
NPAD = 50176
SLICE = NPAD // 16
F = 64
HF = 32
E = 800000
EPAD = 819200
EW = EPAD // 32
EW3 = EPAD // 16
C = 128
C1 = 64
NCH1 = EW // C1
NCH3 = EW3 // C
SLOPE = 0.2
BLK_A = 400
BLK_B = 400

_mesh = plsc.VectorSubcoreMesh(core_axis_name="c", subcore_axis_name="s")
_params = pltpu.CompilerParams(use_tc_tiling_on_sc=False, needs_layout_passes=False)


def _iota16():
    return lax.iota(jnp.int32, 16)


def _zbody(m_ref, d_ref, nt_ref, wm_ref, wd_ref, z_ref, zc_ref):
    zm = jnp.dot(m_ref[...], wm_ref[...], preferred_element_type=jnp.float32)
    zd = jnp.dot(d_ref[...], wd_ref[...], preferred_element_type=jnp.float32)
    z = jnp.where(nt_ref[...] == 0, zm, zd)
    z_ref[...] = z
    zc_ref[...] = jnp.concatenate([z[:, :HF], z[:, HF:]], axis=0)


def _compute_z(m_sim, d_sim, nt, W_m, W_d):
    md = m_sim.shape[1]
    dd = d_sim.shape[1]
    return pl.pallas_call(
        _zbody,
        grid=(N // BLK_A,),
        in_specs=[
            pl.BlockSpec((BLK_A, md), lambda i: (i, 0)),
            pl.BlockSpec((BLK_A, dd), lambda i: (i, 0)),
            pl.BlockSpec((BLK_A, 1), lambda i: (i, 0)),
            pl.BlockSpec((md, F), lambda i: (0, 0)),
            pl.BlockSpec((dd, F), lambda i: (0, 0)),
        ],
        out_specs=[
            pl.BlockSpec((BLK_A, F), lambda i: (i, 0)),
            pl.BlockSpec((2 * BLK_A, HF), lambda i: (i, 0)),
        ],
        out_shape=[
            jax.ShapeDtypeStruct((NPAD, F), jnp.float32),
            jax.ShapeDtypeStruct((2 * NPAD, HF), jnp.float32),
        ],
    )(m_sim, d_sim, nt, W_m, W_d)


@functools.partial(
    pl.kernel,
    out_type=[
        jax.ShapeDtypeStruct((EPAD,), jnp.float32),
        jax.ShapeDtypeStruct((2, NPAD), jnp.float32),
    ],
    mesh=_mesh,
    compiler_params=_params,
    scratch_types=[
        pltpu.VMEM((2, C1), jnp.int32),
        pltpu.VMEM((2, C1), jnp.int32),
        pltpu.VMEM((2, C1, F), jnp.float32),
        pltpu.VMEM((2, C1, F), jnp.float32),
        pltpu.VMEM((C1,), jnp.float32),
        pltpu.VMEM((NPAD,), jnp.float32),
        pltpu.VMEM((SLICE,), jnp.float32),
        pltpu.VMEM((SLICE,), jnp.float32),
        pltpu.VMEM_SHARED((16, NPAD), jnp.float32),
        pltpu.SemaphoreType.DMA,
        pltpu.SemaphoreType.DMA,
        pltpu.SemaphoreType.DMA,
        pltpu.SemaphoreType.DMA,
    ],
)
def _sc_pass1(z_hbm, src_hbm, dst_hbm, e_hbm, emax_hbm,
              src_v, dst_v, zs_v, zd_v, e_v, emax_p, acc_v, tmp_v, sp,
              semm0, semm1, semg0, semg1):
    cid = lax.axis_index("c")
    sid = lax.axis_index("s")
    base = (cid * 16 + sid) * EW
    iota = _iota16()
    semm = (semm0, semm1)
    semg = (semg0, semg1)

    def init_body(i, _):
        emax_p[pl.ds(i * 16, 16)] = jnp.full((16,), -jnp.inf, jnp.float32)
        return 0
    lax.fori_loop(0, NPAD // 16, init_body, 0)

    def meta_issue(cc, b):
        off = base + cc * C1
        pltpu.async_copy(src_hbm.at[pl.ds(off, C1)], src_v.at[b], semm[b])
        pltpu.async_copy(dst_hbm.at[pl.ds(off, C1)], dst_v.at[b], semm[b])

    def meta_wait(b):
        pltpu.make_async_copy(src_hbm.at[pl.ds(0, C1)], src_v.at[b], semm[b]).wait()
        pltpu.make_async_copy(dst_hbm.at[pl.ds(0, C1)], dst_v.at[b], semm[b]).wait()

    def gather_issue(b):
        pltpu.async_copy(z_hbm.at[src_v.at[b]], zs_v.at[b], semg[b])
        pltpu.async_copy(z_hbm.at[dst_v.at[b]], zd_v.at[b], semg[b])

    def gather_wait(b):
        pltpu.make_async_copy(z_hbm.at[pl.ds(0, C1)], zs_v.at[b], semg[b]).wait()
        pltpu.make_async_copy(z_hbm.at[pl.ds(0, C1)], zd_v.at[b], semg[b]).wait()

    meta_issue(0, 0)
    meta_issue(1, 1)
    meta_wait(0)
    gather_issue(0)

    def pair_body(it, _):
        ci = it * 2
        for b in range(2):
            cc = ci + b
            gather_wait(b)

            @pl.when(cc + 2 < NCH1)
            def _():
                meta_issue(cc + 2, b)

            for g in range(C1 // 16):
                rows = iota + (g * 16)

                def dot_body(f, acc):
                    fs = jnp.full((16,), f, jnp.int32)
                    sv = plsc.load_gather(zs_v.at[b], [rows, fs])
                    dv = plsc.load_gather(zd_v.at[b], [rows, fs])
                    return acc + sv * dv
                acc = lax.fori_loop(0, F, dot_body, jnp.zeros((16,), jnp.float32))
                e16 = jnp.where(acc >= 0.0, acc, SLOPE * acc)
                e_v[pl.ds(g * 16, 16)] = e16
                dv16 = dst_v[b, pl.ds(g * 16, 16)]

                def wbody(go):
                    cur = plsc.load_gather(emax_p, [dv16])
                    need = e16 > cur
                    plsc.store_scatter(emax_p, [dv16], jnp.maximum(cur, e16), mask=need)
                    return jnp.any(need)
                lax.while_loop(lambda go: go, wbody, jnp.bool_(True))
            pltpu.sync_copy(e_v, e_hbm.at[pl.ds(base + cc * C1, C1)])

            @pl.when(cc + 1 < NCH1)
            def _():
                meta_wait(1 - b)
                gather_issue(1 - b)
        return 0
    lax.fori_loop(0, NCH1 // 2, pair_body, 0)

    pltpu.sync_copy(emax_p, sp.at[sid])
    plsc.subcore_barrier()
    roff = sid * SLICE
    pltpu.sync_copy(sp.at[0, pl.ds(roff, SLICE)], acc_v)

    def comb_body(t, _):
        pltpu.sync_copy(sp.at[t, pl.ds(roff, SLICE)], tmp_v)

        def vb(i, _):
            s = pl.ds(i * 16, 16)
            acc_v[s] = jnp.maximum(acc_v[s], tmp_v[s])
            return 0
        lax.fori_loop(0, SLICE // 16, vb, 0)
        return 0
    lax.fori_loop(1, 16, comb_body, 0)
    pltpu.sync_copy(acc_v, emax_hbm.at[cid, pl.ds(roff, SLICE)])


@functools.partial(
    pl.kernel,
    out_type=jax.ShapeDtypeStruct((EPAD,), jnp.float32),
    mesh=_mesh,
    compiler_params=_params,
    scratch_types=[
        pltpu.VMEM((C,), jnp.float32),
        pltpu.VMEM((C,), jnp.int32),
        pltpu.VMEM((C,), jnp.float32),
        pltpu.VMEM((NPAD,), jnp.float32),
        pltpu.VMEM((NPAD,), jnp.float32),
        pltpu.VMEM((SLICE,), jnp.float32),
        pltpu.VMEM((SLICE,), jnp.float32),
        pltpu.VMEM_SHARED((NPAD,), jnp.float32),
    ],
)
def _sc_pass2(e_hbm, dst_hbm, emax2_hbm, al_hbm,
              e_v, dst_v, w_v, emax_p, den_p, tmp_v, zero_v, den_sp):
    cid = lax.axis_index("c")
    sid = lax.axis_index("s")
    roff = sid * SLICE

    pltpu.sync_copy(emax2_hbm.at[0], emax_p)

    def mslice(j, _):
        pltpu.sync_copy(emax2_hbm.at[1, pl.ds(j * SLICE, SLICE)], tmp_v)

        def vb(i, _):
            s = pl.ds(i * 16, 16)
            k = pl.ds(j * SLICE + i * 16, 16)
            emax_p[k] = jnp.maximum(emax_p[k], tmp_v[s])
            return 0
        lax.fori_loop(0, SLICE // 16, vb, 0)
        return 0
    lax.fori_loop(0, 16, mslice, 0)

    def zb(i, _):
        zero_v[pl.ds(i * 16, 16)] = jnp.zeros((16,), jnp.float32)
        return 0
    lax.fori_loop(0, SLICE // 16, zb, 0)
    pltpu.sync_copy(zero_v, den_sp.at[pl.ds(roff, SLICE)])
    plsc.subcore_barrier()

    base1 = sid * EW3

    def den_body(ci, _):
        off = base1 + ci * C
        pltpu.sync_copy(e_hbm.at[pl.ds(off, C)], e_v)
        pltpu.sync_copy(dst_hbm.at[pl.ds(off, C)], dst_v)
        for g in range(C // 16):
            s = pl.ds(g * 16, 16)
            m16 = plsc.load_gather(emax_p, [dst_v[s]])
            w_v[s] = jnp.exp(e_v[s] - m16)
        pltpu.sync_copy(w_v, den_sp.at[dst_v], add=True)
        return 0
    lax.fori_loop(0, EW3 // C, den_body, 0)
    plsc.subcore_barrier()

    pltpu.sync_copy(den_sp, den_p)

    def sb(i, _):
        s = pl.ds(i * 16, 16)
        d = den_p[s]
        den_p[s] = jnp.where(d > 0.0, d, 1.0)
        return 0
    lax.fori_loop(0, NPAD // 16, sb, 0)

    base2 = (cid * 16 + sid) * EW

    def al_body(ci, _):
        off = base2 + ci * C
        pltpu.sync_copy(e_hbm.at[pl.ds(off, C)], e_v)
        pltpu.sync_copy(dst_hbm.at[pl.ds(off, C)], dst_v)
        for g in range(C // 16):
            s = pl.ds(g * 16, 16)
            m16 = plsc.load_gather(emax_p, [dst_v[s]])
            d16 = plsc.load_gather(den_p, [dst_v[s]])
            w_v[s] = jnp.exp(e_v[s] - m16) / d16
        pltpu.sync_copy(w_v, al_hbm.at[pl.ds(off, C)])
        return 0
    lax.fori_loop(0, EW // C, al_body, 0)


@functools.partial(
    pl.kernel,
    out_type=jax.ShapeDtypeStruct((2, NPAD, HF), jnp.float32),
    mesh=_mesh,
    compiler_params=_params,
    scratch_types=[
        pltpu.VMEM((2, C), jnp.int32),
        pltpu.VMEM((2, C), jnp.int32),
        pltpu.VMEM((2, C), jnp.float32),
        pltpu.VMEM((2, C, HF), jnp.float32),
        pltpu.VMEM((448, HF), jnp.float32),
        pltpu.VMEM_SHARED((NPAD, HF), jnp.float32),
        pltpu.SemaphoreType.DMA,
        pltpu.SemaphoreType.DMA,
        pltpu.SemaphoreType.DMA,
        pltpu.SemaphoreType.DMA,
    ],
)
def _sc_pass3(zc_hbm, srcx_hbm, dst_hbm, al_hbm, h2_hbm,
              sx_v, dst_v, al_v, zr_v, rb_v, hsp, semm0, semm1, semg0, semg1):
    cid = lax.axis_index("c")
    sid = lax.axis_index("s")
    base = sid * EW3
    iota = _iota16()
    roff = sid * SLICE
    semm = (semm0, semm1)
    semg = (semg0, semg1)

    def zrow2(i, _):
        def zcol(k, _):
            rb_v[i, pl.ds(k * 16, 16)] = jnp.zeros((16,), jnp.float32)
            return 0
        lax.fori_loop(0, HF // 16, zcol, 0)
        return 0
    lax.fori_loop(0, 448, zrow2, 0)

    def zcp(i, _):
        pltpu.sync_copy(rb_v, hsp.at[pl.ds(roff + i * 448, 448)])
        return 0
    lax.fori_loop(0, SLICE // 448, zcp, 0)
    plsc.subcore_barrier()

    def meta_issue(cc, b):
        off = base + cc * C
        pltpu.async_copy(srcx_hbm.at[cid, pl.ds(off, C)], sx_v.at[b], semm[b])
        pltpu.async_copy(dst_hbm.at[pl.ds(off, C)], dst_v.at[b], semm[b])
        pltpu.async_copy(al_hbm.at[pl.ds(off, C)], al_v.at[b], semm[b])

    def meta_wait(b):
        pltpu.make_async_copy(dst_hbm.at[pl.ds(0, C)], sx_v.at[b], semm[b]).wait()
        pltpu.make_async_copy(dst_hbm.at[pl.ds(0, C)], dst_v.at[b], semm[b]).wait()
        pltpu.make_async_copy(al_hbm.at[pl.ds(0, C)], al_v.at[b], semm[b]).wait()

    def gather_issue(b):
        pltpu.async_copy(zc_hbm.at[sx_v.at[b]], zr_v.at[b], semg[b])

    def gather_wait(b):
        pltpu.make_async_copy(zc_hbm.at[pl.ds(0, C)], zr_v.at[b], semg[b]).wait()

    meta_issue(0, 0)
    meta_issue(1, 1)
    meta_wait(0)
    gather_issue(0)

    def pair_body(it, _):
        ci = it * 2
        for b in range(2):
            cc = ci + b
            gather_wait(b)

            @pl.when(cc + 2 < NCH3)
            def _():
                meta_issue(cc + 2, b)

            for g in range(C // 16):
                s = pl.ds(g * 16, 16)
                rows = iota + (g * 16)
                al16 = al_v[b, s]

                def scale_body(f, _):
                    fs = jnp.full((16,), f, jnp.int32)
                    col = plsc.load_gather(zr_v.at[b], [rows, fs])
                    plsc.store_scatter(zr_v.at[b], [rows, fs], col * al16)
                    return 0
                lax.fori_loop(0, HF, scale_body, 0)
            pltpu.sync_copy(zr_v.at[b], hsp.at[dst_v.at[b]], add=True)

            @pl.when(cc + 1 < NCH3)
            def _():
                meta_wait(1 - b)
                gather_issue(1 - b)
        return 0
    lax.fori_loop(0, NCH3 // 2, pair_body, 0)

    plsc.subcore_barrier()

    def outcp(i, _):
        r0 = roff + i * 448
        pltpu.sync_copy(hsp.at[pl.ds(r0, 448)], rb_v)
        pltpu.sync_copy(rb_v, h2_hbm.at[cid, pl.ds(r0, 448)])
        return 0
    lax.fori_loop(0, SLICE // 448, outcp, 0)


def _elu_body(a_ref, b_ref, o_ref):
    h = jnp.concatenate([a_ref[0], b_ref[0]], axis=1)
    o_ref[...] = jnp.where(h > 0, h, jnp.exp(h) - 1.0)


def _elu_merge(h2):
    return pl.pallas_call(
        _elu_body,
        grid=(N // BLK_B,),
        in_specs=[
            pl.BlockSpec((1, BLK_B, HF), lambda i: (0, i, 0)),
            pl.BlockSpec((1, BLK_B, HF), lambda i: (1, i, 0)),
        ],
        out_specs=pl.BlockSpec((BLK_B, F), lambda i: (i, 0)),
        out_shape=jax.ShapeDtypeStruct((N, F), jnp.float32),
    )(h2, h2)


def kernel(m_sim, d_sim, node_type, edge_index, W_m, W_d):
    nt = node_type.astype(jnp.int32).reshape(N, 1)
    src = edge_index[0].astype(jnp.int32)
    dst = edge_index[1].astype(jnp.int32)
    pad_ids = (jnp.arange(EPAD - E, dtype=jnp.int32) % (NPAD - N)) + N
    srcp = jnp.concatenate([src, pad_ids])
    dstp = jnp.concatenate([dst, pad_ids])
    lo_row = srcp + (srcp // BLK_A) * BLK_A
    srcx = jnp.stack([lo_row, lo_row + BLK_A])

    z, zcat = _compute_z(m_sim, d_sim, nt, W_m, W_d)

    e, emax2 = _sc_pass1(z, srcp, dstp)
    alpha = _sc_pass2(e, dstp, emax2)
    h2 = _sc_pass3(zcat, srcx, dstp, alpha)
    return _elu_merge(h2)
